# Initial kernel scaffold; baseline (speedup 1.0000x reference)
import jax
import jax.numpy as jnp
from jax.experimental import pallas as pl


def kernel(x, edge_index, W1l, b1l, W1r, W2l, b2l, W2r, Wlin, blin):
    raise NotImplementedError("write your pallas kernel here")


# speedup vs baseline: 4.3506x; 4.3506x over previous
import functools

import jax
import jax.numpy as jnp
from jax import lax
from jax.experimental import pallas as pl
from jax.experimental.pallas import tpu as pltpu
from jax.experimental.pallas import tpu_sc as plsc

N = 10000
E = 320000
D = 128

NC = 2
NS = 16
NW = NC * NS
CH = 128
NCH = 79
EPW = CH * NCH
EPAD = NW * EPW
NP = 10240
RPT = NP // NS
PAD_DST = NP - 8


_MESH = plsc.VectorSubcoreMesh(core_axis_name="c", subcore_axis_name="s")


def _sc_agg_body(x_hbm, src_hbm, dst_hbm, zrow_hbm,
                 agg_out, src_v, dst_v, rows_v, agg_sh, sem):
  c = lax.axis_index("c")
  s = lax.axis_index("s")
  wid = c * NS + s
  base = s * RPT

  pltpu.sync_copy(src_hbm.at[wid], src_v)
  pltpu.sync_copy(dst_hbm.at[wid], dst_v)
  pltpu.sync_copy(zrow_hbm, agg_sh.at[pl.ds(base, RPT)])
  plsc.subcore_barrier()

  def chunk(j, carry):
    pltpu.async_copy(x_hbm.at[src_v.at[j]], rows_v, sem).wait()
    pltpu.sync_copy(rows_v, agg_sh.at[dst_v.at[j]], add=True)
    return carry

  lax.fori_loop(0, NCH, chunk, 0)
  plsc.subcore_barrier()

  pltpu.sync_copy(agg_sh.at[pl.ds(base, RPT)],
                  agg_out.at[c].at[pl.ds(base, RPT)])


_sc_agg = pl.kernel(
    _sc_agg_body,
    out_type=jax.ShapeDtypeStruct((NC, NP, D), jnp.float32),
    mesh=_MESH,
    scratch_types=[
        pltpu.VMEM((NCH, CH), jnp.int32),
        pltpu.VMEM((NCH, CH), jnp.int32),
        pltpu.VMEM((CH, D), jnp.float32),
        pltpu.VMEM_SHARED((NP, D), jnp.float32),
        pltpu.SemaphoreType.DMA,
    ])


def _sc_cnt_body(dst_hbm, zrow_hbm, ones_hbm,
                 cnt_out, dst_v, ones_v, cnt_sh):
  c = lax.axis_index("c")
  s = lax.axis_index("s")
  wid = c * NS + s
  base = s * RPT

  pltpu.sync_copy(dst_hbm.at[wid], dst_v)
  pltpu.sync_copy(ones_hbm, ones_v)
  pltpu.sync_copy(zrow_hbm, cnt_sh.at[pl.ds(base, RPT)])
  plsc.subcore_barrier()

  def chunk(j, carry):
    pltpu.sync_copy(ones_v, cnt_sh.at[dst_v.at[j]], add=True)
    return carry

  lax.fori_loop(0, NCH, chunk, 0)
  plsc.subcore_barrier()

  pltpu.sync_copy(cnt_sh.at[pl.ds(base, RPT)],
                  cnt_out.at[c].at[pl.ds(base, RPT)])


_sc_cnt = pl.kernel(
    _sc_cnt_body,
    out_type=jax.ShapeDtypeStruct((NC, NP, D), jnp.float32),
    mesh=_MESH,
    scratch_types=[
        pltpu.VMEM((NCH, CH), jnp.int32),
        pltpu.VMEM((CH, D), jnp.float32),
        pltpu.VMEM_SHARED((NP, D), jnp.float32),
    ])

RB = 1000


def _tc_layer1(x_ref, a0_ref, a1_ref, c0_ref, c1_ref,
               w1lt_ref, b1l_ref, w1rt_ref, h1_ref):
  cnt = c0_ref[:, 0:1] + c1_ref[:, 0:1]
  inv = 1.0 / jnp.maximum(cnt, 1.0)
  mean = (a0_ref[...] + a1_ref[...]) * inv
  h1 = (jnp.dot(mean, w1lt_ref[...], preferred_element_type=jnp.float32)
        + b1l_ref[...]
        + jnp.dot(x_ref[...], w1rt_ref[...],
                  preferred_element_type=jnp.float32))
  h1_ref[...] = jnp.maximum(h1, 0.0)


def _tc_layer2(h1_ref, a0_ref, a1_ref, c0_ref, c1_ref,
               w2lt_ref, b2l_ref, w2rt_ref,
               wl1t_ref, wl2t_ref, blin_ref, out_ref):
  cnt = c0_ref[:, 0:1] + c1_ref[:, 0:1]
  inv = 1.0 / jnp.maximum(cnt, 1.0)
  mean = (a0_ref[...] + a1_ref[...]) * inv
  h1 = h1_ref[...]
  h2 = (jnp.dot(mean, w2lt_ref[...], preferred_element_type=jnp.float32)
        + b2l_ref[...]
        + jnp.dot(h1, w2rt_ref[...], preferred_element_type=jnp.float32))
  h2 = jnp.maximum(h2, 0.0)
  out = (jnp.dot(h1, wl1t_ref[...], preferred_element_type=jnp.float32)
         + jnp.dot(h2, wl2t_ref[...], preferred_element_type=jnp.float32)
         + blin_ref[...])
  out_ref[...] = jnp.maximum(out, 0.0)


def _row_spec(width):
  return pl.BlockSpec((RB, width), lambda i: (i, 0))


def _full_spec(shape):
  return pl.BlockSpec(shape, lambda i: tuple(0 for _ in shape))


def kernel(x, edge_index, W1l, b1l, W1r, W2l, b2l, W2r, Wlin, blin):
  src = jnp.concatenate(
      [edge_index[0].astype(jnp.int32),
       jnp.zeros((EPAD - E,), jnp.int32)]).reshape(NW, NCH, CH)
  dst = jnp.concatenate(
      [edge_index[1].astype(jnp.int32),
       jnp.full((EPAD - E,), PAD_DST, jnp.int32)]).reshape(NW, NCH, CH)
  zrow = jnp.zeros((RPT, D), jnp.float32)
  ones = jnp.ones((CH, D), jnp.float32)

  cnt = _sc_cnt(dst, zrow, ones)
  agg1 = _sc_agg(x, src, dst, zrow)
  c0 = cnt[0]
  c1 = cnt[1]

  grid = (N // RB,)
  h1 = pl.pallas_call(
      _tc_layer1,
      grid=grid,
      in_specs=[_row_spec(D), _row_spec(D), _row_spec(D),
                _row_spec(D), _row_spec(D),
                _full_spec((D, D)), _full_spec((D,)), _full_spec((D, D))],
      out_specs=_row_spec(D),
      out_shape=jax.ShapeDtypeStruct((N, D), jnp.float32),
  )(x, agg1[0], agg1[1], c0, c1, W1l.T, b1l, W1r.T)

  agg2 = _sc_agg(h1, src, dst, zrow)

  out = pl.pallas_call(
      _tc_layer2,
      grid=grid,
      in_specs=[_row_spec(D), _row_spec(D), _row_spec(D),
                _row_spec(D), _row_spec(D),
                _full_spec((D, D)), _full_spec((D,)), _full_spec((D, D)),
                _full_spec((D, D)), _full_spec((D, D)), _full_spec((D,))],
      out_specs=_row_spec(D),
      out_shape=jax.ShapeDtypeStruct((N, D), jnp.float32),
  )(h1, agg2[0], agg2[1], c0, c1, W2l.T, b2l, W2r.T,
    Wlin[:, :D].T, Wlin[:, D:].T, blin)
  return out
